# Initial kernel scaffold; baseline (speedup 1.0000x reference)
import jax
import jax.numpy as jnp
from jax.experimental import pallas as pl


def kernel(x, emb, pe):
    raise NotImplementedError("write your pallas kernel here")


# speedup vs baseline: 3.4712x; 3.4712x over previous
import functools

import jax
import jax.numpy as jnp
from jax import lax
from jax.experimental import pallas as pl
from jax.experimental.pallas import tpu as pltpu
from jax.experimental.pallas import tpu_sc as plsc

_NC = 2
_NS = 16
_NW = _NC * _NS
_LANES = 16


def _build_sc_lookup(n_tokens, vocab, d_model, seq_len):
    assert n_tokens % _NW == 0
    per_w = n_tokens // _NW
    chunk_rows = 2
    chunk = chunk_rows * seq_len
    assert per_w % chunk == 0
    n_chunks = per_w // chunk
    n_col = d_model // _LANES

    mesh = plsc.VectorSubcoreMesh(core_axis_name="c", subcore_axis_name="s")

    @functools.partial(
        pl.kernel,
        mesh=mesh,
        compiler_params=pltpu.CompilerParams(use_tc_tiling_on_sc=False),
        out_type=jax.ShapeDtypeStruct((n_tokens, d_model), jnp.float32),
        scratch_types=[
            pltpu.VMEM((seq_len, d_model), jnp.float32),
            pltpu.VMEM((chunk,), jnp.int32),
            pltpu.VMEM((chunk, d_model), jnp.float32),
        ],
    )
    def run(x_hbm, emb_hbm, pe_hbm, out_hbm, pe_v, idx_v, rows_v):
        wid = lax.axis_index("s") * _NC + lax.axis_index("c")
        base = wid * per_w
        pltpu.sync_copy(pe_hbm, pe_v)

        @pl.loop(0, n_chunks)
        def _chunk_loop(ci):
            cbase = base + ci * chunk
            pltpu.sync_copy(x_hbm.at[pl.ds(cbase, chunk)], idx_v)
            pltpu.sync_copy(emb_hbm.at[idx_v], rows_v)

            @pl.loop(0, seq_len)
            def _pe_loop(l):
                pv = [pe_v[l, pl.ds(c * _LANES, _LANES)] for c in range(n_col)]
                for k in range(chunk_rows):
                    for c in range(n_col):
                        plsc.addupdate(
                            rows_v.at[k * seq_len + l, pl.ds(c * _LANES, _LANES)],
                            pv[c],
                        )

            pltpu.sync_copy(rows_v, out_hbm.at[pl.ds(cbase, chunk)])

    return run


def kernel(x, emb, pe):
    batch, seq_len = x.shape
    vocab, d_model = emb.shape
    x_flat = x.reshape(batch * seq_len).astype(jnp.int32)
    pe_block = pe[0, :seq_len].astype(jnp.float32)
    run = _build_sc_lookup(batch * seq_len, vocab, d_model, seq_len)
    out_flat = run(x_flat, emb, pe_block)
    return out_flat.reshape(batch, seq_len, d_model)


# speedup vs baseline: 4.2408x; 1.2217x over previous
import functools

import jax
import jax.numpy as jnp
from jax import lax
from jax.experimental import pallas as pl
from jax.experimental.pallas import tpu as pltpu
from jax.experimental.pallas import tpu_sc as plsc

_NC = 2
_NS = 16
_NW = _NC * _NS
_LANES = 16
_NSLOT = 4


def _build_sc_lookup(n_tokens, vocab, d_model, seq_len):
    assert n_tokens % _NW == 0
    per_w = n_tokens // _NW
    chunk_rows = 2
    chunk = chunk_rows * seq_len
    assert per_w % chunk == 0
    n_chunks = per_w // chunk
    assert n_chunks % _NSLOT == 0 and n_chunks >= 2 * _NSLOT
    n_col = d_model // _LANES

    mesh = plsc.VectorSubcoreMesh(core_axis_name="c", subcore_axis_name="s")

    @functools.partial(
        pl.kernel,
        mesh=mesh,
        compiler_params=pltpu.CompilerParams(use_tc_tiling_on_sc=False),
        out_type=jax.ShapeDtypeStruct((n_tokens, d_model), jnp.float32),
        scratch_types=(
            [pltpu.VMEM((seq_len, d_model), jnp.float32)]
            + [pltpu.VMEM((chunk,), jnp.int32)] * _NSLOT
            + [pltpu.VMEM((chunk, d_model), jnp.float32)] * _NSLOT
            + [pltpu.SemaphoreType.DMA] * (3 * _NSLOT)
        ),
    )
    def run(x_hbm, emb_hbm, pe_hbm, out_hbm, *scr):
        pe_v = scr[0]
        idx_v = scr[1:1 + _NSLOT]
        rows_v = scr[1 + _NSLOT:1 + 2 * _NSLOT]
        isem = scr[1 + 2 * _NSLOT:1 + 3 * _NSLOT]
        gsem = scr[1 + 3 * _NSLOT:1 + 4 * _NSLOT]
        osem = scr[1 + 4 * _NSLOT:1 + 5 * _NSLOT]

        wid = lax.axis_index("s") * _NC + lax.axis_index("c")
        base = wid * per_w

        def idx_copy(b, c):
            return pltpu.make_async_copy(
                x_hbm.at[pl.ds(base + c * chunk, chunk)], idx_v[b], isem[b])

        def gather(b):
            return pltpu.make_async_copy(emb_hbm.at[idx_v[b]], rows_v[b], gsem[b])

        def writeback(b, c):
            return pltpu.make_async_copy(
                rows_v[b], out_hbm.at[pl.ds(base + c * chunk, chunk)], osem[b])

        def pe_add(b):
            @pl.loop(0, seq_len)
            def _pe_loop(l):
                pv = [pe_v[l, pl.ds(cc * _LANES, _LANES)] for cc in range(n_col)]
                for k in range(chunk_rows):
                    for cc in range(n_col):
                        plsc.addupdate(
                            rows_v[b].at[k * seq_len + l,
                                         pl.ds(cc * _LANES, _LANES)],
                            pv[cc],
                        )

        pltpu.sync_copy(pe_hbm, pe_v)

        for b in range(_NSLOT):
            idx_copy(b, b).start()
        for b in range(2):
            idx_copy(b, b).wait()
            gather(b).start()

        @pl.loop(0, n_chunks, step=_NSLOT)
        def _chunk_loop(c0):
            for b in range(_NSLOT):
                c = c0 + b
                b2 = (b + 2) % _NSLOT

                @pl.when(c + 2 < n_chunks)
                def _issue_next_gather():
                    @pl.when(c - 2 >= 0)
                    def _drain_wb():
                        writeback(b2, c - 2).wait()
                    idx_copy(b2, c + 2).wait()
                    gather(b2).start()

                gather(b).wait()
                pe_add(b)
                writeback(b, c).start()

                @pl.when(c + _NSLOT < n_chunks)
                def _prefetch_idx():
                    idx_copy(b, c + _NSLOT).start()

        for b in range(_NSLOT):
            writeback(b, n_chunks - _NSLOT + b).wait()

    return run


def kernel(x, emb, pe):
    batch, seq_len = x.shape
    vocab, d_model = emb.shape
    x_flat = x.reshape(batch * seq_len).astype(jnp.int32)
    pe_block = pe[0, :seq_len].astype(jnp.float32)
    run = _build_sc_lookup(batch * seq_len, vocab, d_model, seq_len)
    out_flat = run(x_flat, emb, pe_block)
    return out_flat.reshape(batch, seq_len, d_model)
